# baseline (speedup 1.0000x reference)
import functools

import jax
import jax.numpy as jnp
from jax import lax
from jax.experimental import pallas as pl
from jax.experimental.pallas import tpu as pltpu
from jax.experimental.pallas import tpu_sc as plsc

NC = 2
NS = 16
NW = NC * NS
L = 16
NPER = 128


def _make_kernel(B, D, NM):
    bpw = B // NW
    n_chunk = bpw // NPER
    grp_per_chunk = NPER // L

    mesh = plsc.VectorSubcoreMesh(core_axis_name="c", subcore_axis_name="s")

    @functools.partial(
        pl.kernel,
        out_type=jax.ShapeDtypeStruct((B * 3,), jnp.float32),
        mesh=mesh,
        compiler_params=pltpu.CompilerParams(
            needs_layout_passes=False, use_tc_tiling_on_sc=False),
        scratch_types=[
            pltpu.VMEM((n_chunk, 3, NPER), jnp.int32),
            pltpu.VMEM((bpw, D), jnp.float32),
            pltpu.VMEM((NM, D), jnp.float32),
            pltpu.VMEM_SHARED((NM, D), jnp.float32),
            pltpu.VMEM((n_chunk, NPER), jnp.float32),
            pltpu.VMEM((bpw * 3,), jnp.float32),
            pltpu.SemaphoreType.DMA((4,)),
        ],
    )
    def btd_kernel(idx_hbm, u_hbm, v_hbm, ll_hbm, out_hbm,
                   idx_v, u_r, v_l, v_sh, ll_v, out_v, sems):
        cid = lax.axis_index("c")
        sid = lax.axis_index("s")
        wid = sid * NC + cid
        base = wid * bpw

        @pl.when(sid == 0)
        def _():
            pltpu.sync_copy(v_hbm, v_sh)

        pltpu.sync_copy(idx_hbm.at[pl.ds(wid * n_chunk, n_chunk)], idx_v)

        copies = []
        for ch in range(n_chunk):
            dsl = pl.ds(ch * NPER, NPER)
            copies.append(
                (pltpu.async_copy(u_hbm.at[idx_v.at[ch, 0]],
                                  u_r.at[dsl], sems.at[ch]),
                 pltpu.async_copy(ll_hbm.at[idx_v.at[ch, 0]],
                                  ll_v.at[ch], sems.at[ch])))

        plsc.subcore_barrier()
        pltpu.sync_copy(v_sh, v_l)

        one16 = jnp.full((L,), 1, jnp.int32)
        two16 = jnp.full((L,), 2, jnp.int32)

        for ch in range(n_chunk):
            for cp in copies[ch]:
                cp.wait()

            @pl.loop(0, grp_per_chunk)
            def _(gg, _ch=ch):
                chv = jnp.full((L,), _ch, jnp.int32)
                pos = gg * L + lax.iota(jnp.int32, L)
                rows = _ch * NPER + pos
                jv = plsc.load_gather(idx_v, [chv, one16, pos])
                kv = plsc.load_gather(idx_v, [chv, two16, pos])
                llv = plsc.load_gather(ll_v, [chv, pos])
                accj = jnp.zeros((L,), jnp.float32)
                acck = jnp.zeros((L,), jnp.float32)
                for d in range(D):
                    dv = jnp.full((L,), d, jnp.int32)
                    uu = plsc.load_gather(u_r, [rows, dv])
                    vj = plsc.load_gather(v_l, [jv, dv])
                    vk = plsc.load_gather(v_l, [kv, dv])
                    accj = accj + uu * vj
                    acck = acck + uu * vk
                tie = llv + 0.5 * (accj + acck)
                rows3 = rows * 3
                plsc.store_scatter(out_v, [rows3], tie)
                plsc.store_scatter(out_v, [rows3 + 1], accj)
                plsc.store_scatter(out_v, [rows3 + 2], acck)

        pltpu.sync_copy(out_v, out_hbm.at[pl.ds(base * 3, bpw * 3)])

    return btd_kernel


@jax.jit
def kernel(c, i, j, k, u, v, log_lambda):
    B = c.shape[0]
    D = u.shape[1]
    NM = v.shape[0]
    run = _make_kernel(B, D, NM)
    row_idx = c.astype(jnp.int32) * NM + i.astype(jnp.int32)
    idx = jnp.stack([row_idx.reshape(B // NPER, NPER),
                     j.astype(jnp.int32).reshape(B // NPER, NPER),
                     k.astype(jnp.int32).reshape(B // NPER, NPER)], axis=1)
    return run(idx, u, v, log_lambda.reshape(-1)).reshape(B, 3)


# speedup vs baseline: 1.0226x; 1.0226x over previous
import functools

import jax
import jax.numpy as jnp
from jax import lax
from jax.experimental import pallas as pl
from jax.experimental.pallas import tpu as pltpu
from jax.experimental.pallas import tpu_sc as plsc

NC = 2
NS = 16
NW = NC * NS
L = 16
NPER = 128


def _make_kernel(B, D, NM):
    bpw = B // NW
    n_chunk = bpw // NPER
    grp_per_chunk = NPER // L

    mesh = plsc.VectorSubcoreMesh(core_axis_name="c", subcore_axis_name="s")

    @functools.partial(
        pl.kernel,
        out_type=jax.ShapeDtypeStruct((B, 3), jnp.float32),
        mesh=mesh,
        compiler_params=pltpu.CompilerParams(
            needs_layout_passes=False, use_tc_tiling_on_sc=False),
        scratch_types=[
            pltpu.VMEM((n_chunk, 3, NPER), jnp.int32),
            pltpu.VMEM((bpw, D), jnp.float32),
            pltpu.VMEM((NM, D), jnp.float32),
            pltpu.VMEM_SHARED((NM, D), jnp.float32),
            pltpu.VMEM((n_chunk, NPER), jnp.float32),
            pltpu.VMEM((bpw, 3), jnp.float32),
            pltpu.SemaphoreType.DMA((4,)),
        ],
    )
    def btd_kernel(idx_hbm, u_hbm, v_hbm, ll_hbm, out_hbm,
                   idx_v, u_r, v_l, v_sh, ll_v, out_v, sems):
        cid = lax.axis_index("c")
        sid = lax.axis_index("s")
        wid = sid * NC + cid
        base = wid * bpw

        @pl.when(sid == 0)
        def _():
            pltpu.sync_copy(v_hbm, v_sh)

        pltpu.sync_copy(idx_hbm.at[pl.ds(wid * n_chunk, n_chunk)], idx_v)

        copies = []
        for ch in range(n_chunk):
            dsl = pl.ds(ch * NPER, NPER)
            copies.append(
                (pltpu.async_copy(u_hbm.at[idx_v.at[ch, 0]],
                                  u_r.at[dsl], sems.at[ch]),
                 pltpu.async_copy(ll_hbm.at[idx_v.at[ch, 0]],
                                  ll_v.at[ch], sems.at[ch])))

        plsc.subcore_barrier()
        pltpu.sync_copy(v_sh, v_l)

        one16 = jnp.full((L,), 1, jnp.int32)
        two16 = jnp.full((L,), 2, jnp.int32)

        for ch in range(n_chunk):
            for cp in copies[ch]:
                cp.wait()

            @pl.loop(0, grp_per_chunk)
            def _(gg, _ch=ch):
                chv = jnp.full((L,), _ch, jnp.int32)
                pos = gg * L + lax.iota(jnp.int32, L)
                rows = _ch * NPER + pos
                jv = plsc.load_gather(idx_v, [chv, one16, pos])
                kv = plsc.load_gather(idx_v, [chv, two16, pos])
                llv = plsc.load_gather(ll_v, [chv, pos])
                accj = jnp.zeros((L,), jnp.float32)
                acck = jnp.zeros((L,), jnp.float32)
                for d in range(D):
                    dv = jnp.full((L,), d, jnp.int32)
                    uu = plsc.load_gather(u_r, [rows, dv])
                    vj = plsc.load_gather(v_l, [jv, dv])
                    vk = plsc.load_gather(v_l, [kv, dv])
                    accj = accj + uu * vj
                    acck = acck + uu * vk
                tie = llv + 0.5 * (accj + acck)
                zero16 = jnp.zeros((L,), jnp.int32)
                plsc.store_scatter(out_v, [rows, zero16], tie)
                plsc.store_scatter(out_v, [rows, zero16 + 1], accj)
                plsc.store_scatter(out_v, [rows, zero16 + 2], acck)

        pltpu.sync_copy(out_v, out_hbm.at[pl.ds(base, bpw)])

    return btd_kernel


@jax.jit
def kernel(c, i, j, k, u, v, log_lambda):
    B = c.shape[0]
    D = u.shape[1]
    NM = v.shape[0]
    run = _make_kernel(B, D, NM)
    row_idx = c.astype(jnp.int32) * NM + i.astype(jnp.int32)
    idx = jnp.stack([row_idx.reshape(B // NPER, NPER),
                     j.astype(jnp.int32).reshape(B // NPER, NPER),
                     k.astype(jnp.int32).reshape(B // NPER, NPER)], axis=1)
    return run(idx, u, v, log_lambda.reshape(-1))
